# baseline (speedup 1.0000x reference)
import jax
import jax.numpy as jnp
from jax.experimental import pallas as pl
from jax.experimental.pallas import tpu as pltpu


def _body(x_ref, w_ref, b_ref, o_ref, xp0_ref, xp1_ref, xcat_ref):
    for t, xpt in enumerate((xp0_ref, xp1_ref)):
        xpt[0:1] = jnp.zeros((1, 66, 128), jnp.float32)
        xpt[65:66] = jnp.zeros((1, 66, 128), jnp.float32)
        xpt[1:65, 0:1] = jnp.zeros((64, 1, 128), jnp.float32)
        xpt[1:65, 65:66] = jnp.zeros((64, 1, 128), jnp.float32)
        xpt[1:65, 1:65] = x_ref[0, :, :, t * 128:(t + 1) * 128].astype(
            jnp.float32)
    for g, (si, sj) in enumerate(((0, 0), (0, 1), (1, 0), (1, 1))):
        for kh in range(3):
            for kw in range(3):
                j = kh * 3 + kw
                for t, xpt in enumerate((xp0_ref, xp1_ref)):
                    xs = xpt[pl.ds(2 * si + kh, 16, 4),
                             pl.ds(2 * sj + kw, 16, 4), :]
                    xcat_ref[g * 256:(g + 1) * 256,
                             j * 256 + t * 128:j * 256 + (t + 1) * 128] = (
                        xs.reshape(256, 128).astype(jnp.bfloat16))
    acc = jnp.dot(xcat_ref[...], w_ref[...],
                  preferred_element_type=jnp.float32)
    acc = acc + b_ref[0, :][None, :]
    o_ref[0] = acc.T


def kernel(x, weight, bias):
    xt = jnp.transpose(x.astype(jnp.bfloat16), (0, 2, 3, 1))
    wc = jnp.transpose(weight, (2, 3, 1, 0)).reshape(2304, 256)
    wc = wc.astype(jnp.bfloat16)
    b2 = bias.reshape(1, 256)
    return pl.pallas_call(
        _body,
        out_shape=jax.ShapeDtypeStruct((16, 256, 1024), jnp.float32),
        grid=(16,),
        in_specs=[
            pl.BlockSpec((1, 64, 64, 256), lambda i: (i, 0, 0, 0)),
            pl.BlockSpec((2304, 256), lambda i: (0, 0)),
            pl.BlockSpec((1, 256), lambda i: (0, 0)),
        ],
        out_specs=pl.BlockSpec((1, 256, 1024), lambda i: (i, 0, 0)),
        scratch_shapes=[
            pltpu.VMEM((66, 66, 128), jnp.float32),
            pltpu.VMEM((66, 66, 128), jnp.float32),
            pltpu.VMEM((1024, 2304), jnp.bfloat16),
        ],
        compiler_params=pltpu.CompilerParams(
            dimension_semantics=("parallel",),
            vmem_limit_bytes=52 * 1024 * 1024,
        ),
        name="conv2d_orion",
    )(xt, wc, b2)


# speedup vs baseline: 1.7176x; 1.7176x over previous
import jax
import jax.numpy as jnp
from jax.experimental import pallas as pl
from jax.experimental.pallas import tpu as pltpu


def _body(x_hbm, w_ref, b_ref, o_ref, xp_ref, xcat_ref, sem):
    i = pl.program_id(0)
    slot = jax.lax.rem(i, 2)
    nslot = jax.lax.rem(i + 1, 2)

    def start(batch, s):
        for t in range(2):
            pltpu.make_async_copy(
                x_hbm.at[batch].at[:, :, t * 128:(t + 1) * 128],
                xp_ref.at[s, t].at[1:65, 1:65, :],
                sem.at[s, t],
            ).start()

    @pl.when(i == 0)
    def _():
        for s in range(2):
            for t in range(2):
                xp_ref[s, t, 0:1] = jnp.zeros((1, 66, 128), jnp.float32)
                xp_ref[s, t, 65:66] = jnp.zeros((1, 66, 128), jnp.float32)
                xp_ref[s, t, 1:65, 0:1] = jnp.zeros((64, 1, 128), jnp.float32)
                xp_ref[s, t, 1:65, 65:66] = jnp.zeros((64, 1, 128),
                                                      jnp.float32)
        start(i, slot)

    @pl.when(i + 1 < 16)
    def _():
        start(i + 1, nslot)

    for t in range(2):
        pltpu.make_async_copy(
            x_hbm.at[0].at[:, :, t * 128:(t + 1) * 128],
            xp_ref.at[slot, t].at[1:65, 1:65, :],
            sem.at[slot, t],
        ).wait()

    xp0 = xp_ref.at[slot, 0]
    xp1 = xp_ref.at[slot, 1]
    for g, (si, sj) in enumerate(((0, 0), (0, 1), (1, 0), (1, 1))):
        for kh in range(3):
            for kw in range(3):
                j = kh * 3 + kw
                for t, xpt in enumerate((xp0, xp1)):
                    xs = xpt[pl.ds(2 * si + kh, 16, 4),
                             pl.ds(2 * sj + kw, 16, 4), :]
                    xcat_ref[g * 256:(g + 1) * 256,
                             j * 256 + t * 128:j * 256 + (t + 1) * 128] = (
                        xs.reshape(256, 128).astype(jnp.bfloat16))
    acc = jnp.dot(xcat_ref[...], w_ref[...],
                  preferred_element_type=jnp.float32)
    acc = acc + b_ref[0, :][None, :]
    o_ref[0] = acc.T


def kernel(x, weight, bias):
    xt = jnp.transpose(x, (0, 2, 3, 1))
    wc = jnp.transpose(weight, (2, 3, 1, 0)).reshape(2304, 256)
    wc = wc.astype(jnp.bfloat16)
    b2 = bias.reshape(1, 256)
    return pl.pallas_call(
        _body,
        out_shape=jax.ShapeDtypeStruct((16, 256, 1024), jnp.float32),
        grid=(16,),
        in_specs=[
            pl.BlockSpec(memory_space=pl.ANY),
            pl.BlockSpec((2304, 256), lambda i: (0, 0)),
            pl.BlockSpec((1, 256), lambda i: (0, 0)),
        ],
        out_specs=pl.BlockSpec((1, 256, 1024), lambda i: (i, 0, 0)),
        scratch_shapes=[
            pltpu.VMEM((2, 2, 66, 66, 128), jnp.float32),
            pltpu.VMEM((1024, 2304), jnp.bfloat16),
            pltpu.SemaphoreType.DMA((2, 2)),
        ],
        compiler_params=pltpu.CompilerParams(
            dimension_semantics=("arbitrary",),
            vmem_limit_bytes=52 * 1024 * 1024,
        ),
        name="conv2d_orion",
    )(xt, wc, b2)
